# baseline (speedup 1.0000x reference)
import functools

import jax
import jax.numpy as jnp
from jax import lax
from jax.experimental import pallas as pl
from jax.experimental.pallas import tpu as pltpu
from jax.experimental.pallas import tpu_sc as plsc

NUM_CORES = 1
NUM_SUBCORES = 16
NUM_WORKERS = NUM_CORES * NUM_SUBCORES


@functools.lru_cache(maxsize=None)
def _make_gather(batch: int):
    per_w = batch // NUM_WORKERS
    half = per_w // 2
    assert per_w % 2 == 0 and half % 8 == 0
    mesh = plsc.VectorSubcoreMesh(core_axis_name="c", subcore_axis_name="s",
                                  num_cores=NUM_CORES)

    @functools.partial(
        pl.kernel,
        mesh=mesh,
        out_type=jax.ShapeDtypeStruct((batch,), jnp.float32),
        scratch_types=[
            pltpu.VMEM((per_w,), jnp.int32),
            pltpu.VMEM((per_w,), jnp.float32),
            pltpu.SemaphoreType.DMA,
            pltpu.SemaphoreType.DMA,
            pltpu.SemaphoreType.DMA,
            pltpu.SemaphoreType.DMA,
        ],
    )
    def gather_kernel(table_hbm, idx_hbm, out_hbm, idx_v, rows_v,
                      si, sg0, sg1, so):
        wid = lax.axis_index("s")
        base = wid * per_w
        pltpu.async_copy(idx_hbm.at[pl.ds(base, per_w)], idx_v, si).wait()
        g0 = pltpu.async_copy(table_hbm.at[idx_v.at[pl.ds(0, half)]],
                              rows_v.at[pl.ds(0, half)], sg0)
        g1 = pltpu.async_copy(table_hbm.at[idx_v.at[pl.ds(half, half)]],
                              rows_v.at[pl.ds(half, half)], sg1)
        g0.wait()
        o0 = pltpu.async_copy(rows_v.at[pl.ds(0, half)],
                              out_hbm.at[pl.ds(base, half)], so)
        g1.wait()
        o1 = pltpu.async_copy(rows_v.at[pl.ds(half, half)],
                              out_hbm.at[pl.ds(base + half, half)], so)
        o0.wait()
        o1.wait()

    return gather_kernel


def kernel(table, e):
    batch = e.shape[0]
    flat_table = table.reshape(-1)
    idx = e.astype(jnp.int32)
    out = _make_gather(batch)(flat_table, idx)
    return out.reshape(batch, 1)


# speedup vs baseline: 1.0013x; 1.0013x over previous
import functools

import jax
import jax.numpy as jnp
from jax import lax
from jax.experimental import pallas as pl
from jax.experimental.pallas import tpu as pltpu
from jax.experimental.pallas import tpu_sc as plsc

NUM_CORES = 1
NUM_SUBCORES = 16
NUM_WORKERS = NUM_CORES * NUM_SUBCORES


@functools.lru_cache(maxsize=None)
def _make_gather(batch: int):
    nchunk = 4
    per_w = batch // NUM_WORKERS
    chunk = per_w // nchunk
    assert per_w % nchunk == 0 and chunk % 8 == 0
    mesh = plsc.VectorSubcoreMesh(core_axis_name="c", subcore_axis_name="s",
                                  num_cores=NUM_CORES)

    @functools.partial(
        pl.kernel,
        mesh=mesh,
        out_type=jax.ShapeDtypeStruct((batch,), jnp.float32),
        scratch_types=(
            [pltpu.VMEM((per_w,), jnp.int32),
             pltpu.VMEM((per_w,), jnp.float32),
             pltpu.SemaphoreType.DMA,
             pltpu.SemaphoreType.DMA]
            + [pltpu.SemaphoreType.DMA] * nchunk
        ),
    )
    def gather_kernel(table_hbm, idx_hbm, out_hbm, idx_v, rows_v,
                      si, so, *sg):
        wid = lax.axis_index("s")
        base = wid * per_w
        pltpu.async_copy(idx_hbm.at[pl.ds(base, per_w)], idx_v, si).wait()
        gathers = [
            pltpu.async_copy(table_hbm.at[idx_v.at[pl.ds(j * chunk, chunk)]],
                             rows_v.at[pl.ds(j * chunk, chunk)], sg[j])
            for j in range(nchunk)
        ]
        stores = []
        for j in range(nchunk):
            gathers[j].wait()
            stores.append(
                pltpu.async_copy(rows_v.at[pl.ds(j * chunk, chunk)],
                                 out_hbm.at[pl.ds(base + j * chunk, chunk)], so))
        for s in stores:
            s.wait()

    return gather_kernel


def kernel(table, e):
    batch = e.shape[0]
    flat_table = table.reshape(-1)
    idx = e.astype(jnp.int32)
    out = _make_gather(batch)(flat_table, idx)
    return out.reshape(batch, 1)
